# baseline (speedup 1.0000x reference)
import functools

import jax
import jax.numpy as jnp
from jax import lax
from jax.experimental import pallas as pl
from jax.experimental.pallas import tpu as pltpu
from jax.experimental.pallas import tpu_sc as plsc

_VOCAB = 1000
_LSE_PAD = 1008
_NC = 2
_NS = 16
_NW = _NC * _NS
_L = 16
_CH = 50
_CHP = 56
_CHI = 64
_NG = 4


def _lse_body(table_ref, out_ref):
    x = table_ref[...]
    m = jnp.max(x, axis=1)
    s = jnp.sum(jnp.exp(x - m[:, None]), axis=1)
    out_ref[...] = m + jnp.log(s)


def _compute_lse(table):
    return pl.pallas_call(
        _lse_body,
        out_shape=jax.ShapeDtypeStruct((table.shape[0],), jnp.float32),
    )(table)


def _sc_body(table_hbm, idx_hbm, tgt_hbm, lse_hbm,
             logits_hbm, part_hbm,
             rows_v, idx_v, tgt_v, lse_v, acc_v,
             g0, g1, s0, s1):
    wid = lax.axis_index("s") * _NC + lax.axis_index("c")
    batw = idx_hbm.shape[0] // _NW
    nch = batw
    bat0 = wid * batw

    pltpu.sync_copy(idx_hbm.at[pl.ds(bat0, batw)], idx_v)
    pltpu.sync_copy(tgt_hbm.at[pl.ds(bat0, batw)], tgt_v)
    pltpu.sync_copy(lse_hbm, lse_v)
    acc_v[...] = jnp.zeros((_L,), jnp.float32)

    gsems = (g0, g1)
    ssems = (s0, s1)

    def gather_desc(c, b):
        return pltpu.make_async_copy(
            table_hbm.at[idx_v.at[c, pl.ds(0, _CHP)]], rows_v.at[b], gsems[b])

    def scatter_desc(c, b):
        return pltpu.make_async_copy(
            rows_v.at[b, pl.ds(0, _CH)], logits_hbm.at[bat0 + c], ssems[b])

    def loss_chunk(c, b):
        for g in range(_NG):
            valid = min(_L, _CH - g * _L)
            idxg = idx_v[c, pl.ds(g * _L, _L)]
            tgtg = tgt_v[c, pl.ds(g * _L, _L)]
            rid = lax.iota(jnp.int32, _L) + g * _L
            if valid == _L:
                picked = plsc.load_gather(rows_v.at[b], [rid, tgtg])
                lsev = plsc.load_gather(lse_v, [idxg])
                acc_v[...] = acc_v[...] + (lsev - picked)
            else:
                m = lax.iota(jnp.int32, _L) < valid
                picked = plsc.load_gather(rows_v.at[b], [rid, tgtg], mask=m)
                lsev = plsc.load_gather(lse_v, [idxg], mask=m)
                acc_v[...] = acc_v[...] + jnp.where(
                    m, lsev - picked, jnp.zeros((_L,), jnp.float32))

    gather_desc(0, 0).start()
    gather_desc(1, 1).start()

    def outer(t, carry):
        for b in range(2):
            c = t * 2 + b
            gather_desc(c, b).wait()
            loss_chunk(c, b)
            scatter_desc(c, b).start()

            @pl.when(c + 2 < nch)
            def _():
                scatter_desc(c, b).wait()
                gather_desc(c + 2, b).start()
        return carry

    lax.fori_loop(0, nch // 2, outer, None)
    scatter_desc(nch - 2, 0).wait()
    scatter_desc(nch - 1, 1).wait()
    pltpu.sync_copy(acc_v, part_hbm.at[wid])


def _sc_gather_loss(table, idx_p, tgt_p, lse_p):
    nbatch = idx_p.shape[0]
    batw = nbatch // _NW
    call = pl.kernel(
        _sc_body,
        out_type=[
            jax.ShapeDtypeStruct((nbatch, _CH, _VOCAB), jnp.float32),
            jax.ShapeDtypeStruct((_NW, _L), jnp.float32),
        ],
        mesh=plsc.VectorSubcoreMesh(core_axis_name="c", subcore_axis_name="s"),
        compiler_params=pltpu.CompilerParams(
            use_tc_tiling_on_sc=False, needs_layout_passes=False),
        scratch_types=[
            pltpu.VMEM((2, _CHP, _VOCAB), jnp.float32),
            pltpu.VMEM((batw, _CHI), jnp.int32),
            pltpu.VMEM((batw, _CHI), jnp.int32),
            pltpu.VMEM((_LSE_PAD,), jnp.float32),
            pltpu.VMEM((_L,), jnp.float32),
            pltpu.SemaphoreType.DMA,
            pltpu.SemaphoreType.DMA,
            pltpu.SemaphoreType.DMA,
            pltpu.SemaphoreType.DMA,
        ],
    )
    return call(table, idx_p, tgt_p, lse_p)


def kernel(idx, targets, table):
    lse = _compute_lse(table)
    lse_p = jnp.pad(lse, (0, _LSE_PAD - _VOCAB))
    idx_p = jnp.pad(idx, ((0, 0), (0, _CHI - _CH)))
    tgt_p = jnp.pad(targets, ((0, 0), (0, _CHI - _CH)))
    logits, partials = _sc_gather_loss(table, idx_p, tgt_p, lse_p)
    loss = jnp.sum(partials) / jnp.float32(idx.shape[0] * idx.shape[1])
    return logits, loss


# speedup vs baseline: 1.5075x; 1.5075x over previous
import functools

import jax
import jax.numpy as jnp
from jax import lax
from jax.experimental import pallas as pl
from jax.experimental.pallas import tpu as pltpu
from jax.experimental.pallas import tpu_sc as plsc

_VOCAB = 1000
_LSE_PAD = 1008
_NC = 2
_NS = 16
_NW = _NC * _NS
_L = 16
_CH = 16
_NSPLIT = 2


def _lse_body(table_ref, out_ref):
    x = table_ref[...]
    m = jnp.max(x, axis=1)
    s = jnp.sum(jnp.exp(x - m[:, None]), axis=1)
    out_ref[...] = m + jnp.log(s)


def _compute_lse(table):
    return pl.pallas_call(
        _lse_body,
        out_shape=jax.ShapeDtypeStruct((table.shape[0],), jnp.float32),
    )(table)


def _sc_body(table_hbm, idx_hbm, tgt_hbm, lse_hbm,
             logits_hbm, part_hbm,
             rows_v, idx_v, tgt_v, lse_v, acc_v, table_sp,
             g0, g1, s0, s1):
    wid = lax.axis_index("s") * _NC + lax.axis_index("c")
    tokw = idx_hbm.shape[0] // _NW
    nch = tokw // _CH
    base = wid * tokw

    @pl.when(lax.axis_index("s") == 0)
    def _():
        pltpu.sync_copy(table_hbm, table_sp)

    pltpu.sync_copy(idx_hbm.at[pl.ds(base, tokw)], idx_v)
    pltpu.sync_copy(tgt_hbm.at[pl.ds(base, tokw)], tgt_v)
    pltpu.sync_copy(lse_hbm, lse_v)
    acc_v[...] = jnp.zeros((_L,), jnp.float32)
    plsc.subcore_barrier()

    gsems = (g0, g1)
    ssems = (s0, s1)

    def gather_desc(c, b):
        return pltpu.make_async_copy(
            table_sp.at[idx_v.at[pl.ds(c * _CH, _CH)]],
            rows_v.at[b], gsems[b])

    def scatter_desc(c, b):
        return pltpu.make_async_copy(
            rows_v.at[b], logits_hbm.at[pl.ds(base + c * _CH, _CH)], ssems[b])

    def loss_chunk(c, b):
        for g in range(_CH // _L):
            off = c * _CH + g * _L
            idxg = idx_v[pl.ds(off, _L)]
            tgtg = tgt_v[pl.ds(off, _L)]
            rid = lax.iota(jnp.int32, _L) + g * _L
            picked = plsc.load_gather(rows_v.at[b], [rid, tgtg])
            lsev = plsc.load_gather(lse_v, [idxg])
            acc_v[...] = acc_v[...] + (lsev - picked)

    gather_desc(0, 0).start()
    gather_desc(1, 1).start()

    def outer(t, carry):
        for b in range(2):
            c = t * 2 + b
            gather_desc(c, b).wait()
            loss_chunk(c, b)
            scatter_desc(c, b).start()

            @pl.when(c + 2 < nch)
            def _():
                scatter_desc(c, b).wait()
                gather_desc(c + 2, b).start()
        return carry

    lax.fori_loop(0, nch // 2, outer, None)
    scatter_desc(nch - 2, 0).wait()
    scatter_desc(nch - 1, 1).wait()
    pltpu.sync_copy(acc_v, part_hbm.at[pl.ds(wid * _L, _L)])


def _sc_gather_loss(table, idx_f, tgt_f, lse_p):
    n = idx_f.shape[0]
    tokw = n // _NW
    call = pl.kernel(
        _sc_body,
        out_type=[
            jax.ShapeDtypeStruct((n, _VOCAB), jnp.float32),
            jax.ShapeDtypeStruct((_NW * _L,), jnp.float32),
        ],
        mesh=plsc.VectorSubcoreMesh(core_axis_name="c", subcore_axis_name="s"),
        compiler_params=pltpu.CompilerParams(
            use_tc_tiling_on_sc=False, needs_layout_passes=False),
        scratch_types=[
            pltpu.VMEM((2, _CH, _VOCAB), jnp.float32),
            pltpu.VMEM((tokw,), jnp.int32),
            pltpu.VMEM((tokw,), jnp.int32),
            pltpu.VMEM((_LSE_PAD,), jnp.float32),
            pltpu.VMEM((_L,), jnp.float32),
            pltpu.VMEM_SHARED((_VOCAB, _VOCAB), jnp.float32),
            pltpu.SemaphoreType.DMA,
            pltpu.SemaphoreType.DMA,
            pltpu.SemaphoreType.DMA,
            pltpu.SemaphoreType.DMA,
        ],
    )
    return call(table, idx_f, tgt_f, lse_p)


def kernel(idx, targets, table):
    nbat, seq = idx.shape
    lse = _compute_lse(table)
    lse_p = jnp.pad(lse, (0, _LSE_PAD - _VOCAB))
    idx_f = idx.reshape(-1)
    tgt_f = targets.reshape(-1)
    n = idx_f.shape[0]
    half = n // _NSPLIT
    bh = nbat // _NSPLIT
    parts = []
    halves = []
    for s in range(_NSPLIT):
        lg, pt = _sc_gather_loss(
            table,
            lax.dynamic_slice_in_dim(idx_f, s * half, half),
            lax.dynamic_slice_in_dim(tgt_f, s * half, half),
            lse_p)
        halves.append(lg.reshape(bh, seq, _VOCAB))
        parts.append(pt)
    logits = jnp.concatenate(halves, axis=0)
    loss = jnp.sum(jnp.stack(parts)) / jnp.float32(n)
    return logits, loss


# speedup vs baseline: 1.9294x; 1.2799x over previous
import functools

import jax
import jax.numpy as jnp
from jax import lax
from jax.experimental import pallas as pl
from jax.experimental.pallas import tpu as pltpu
from jax.experimental.pallas import tpu_sc as plsc

_VOCAB = 1000
_LSE_PAD = 1008
_NC = 2
_NS = 16
_NW = _NC * _NS
_L = 16
_SEQ = 50
_SEQP = 56
_PITCH = 64
_CH = 8


def _lse_body(table_ref, out_ref):
    x = table_ref[...]
    m = jnp.max(x, axis=1)
    s = jnp.sum(jnp.exp(x - m[:, None]), axis=1)
    out_ref[...] = m + jnp.log(s)


def _compute_lse(table):
    return pl.pallas_call(
        _lse_body,
        out_shape=jax.ShapeDtypeStruct((table.shape[0],), jnp.float32),
    )(table)


def _sc_body(table_hbm, tflat_hbm, idx_hbm, tgt_hbm, lse_hbm,
             rows_hbm, part_hbm,
             rows_v, idx_v, tgt_v, fidx_v, picks_v, lse_v, acc_v, table_sp,
             g0, g1, s0, s1, psem):
    wid = lax.axis_index("s") * _NC + lax.axis_index("c")
    batw = (rows_hbm.shape[0] // _SEQP) // _NW
    npick = batw * _PITCH
    ib0 = wid * npick
    rb0 = wid * batw * _SEQP
    ncht = batw * (_SEQP // _CH)

    @pl.when(lax.axis_index("s") == 0)
    def _():
        pltpu.sync_copy(table_hbm, table_sp)

    pltpu.sync_copy(idx_hbm.at[pl.ds(ib0, npick)], idx_v)
    pltpu.sync_copy(tgt_hbm.at[pl.ds(ib0, npick)], tgt_v)
    pltpu.sync_copy(lse_hbm, lse_v)
    acc_v[...] = jnp.zeros((_L,), jnp.float32)
    plsc.subcore_barrier()

    def build(i, carry):
        off = i * _L
        fidx_v[pl.ds(off, _L)] = (
            idx_v[pl.ds(off, _L)] * 1024 + tgt_v[pl.ds(off, _L)])
        return carry

    lax.fori_loop(0, npick // _L, build, None)

    def pick_desc(k):
        return pltpu.make_async_copy(
            tflat_hbm.at[fidx_v.at[pl.ds(k * 128, 128)]],
            picks_v.at[pl.ds(k * 128, 128)], psem)

    nk = npick // 128
    for k in range(nk):
        pick_desc(k).start()

    gsems = (g0, g1)
    ssems = (s0, s1)

    def gather_desc(c, b):
        bat_l = c // (_SEQP // _CH)
        s = c - bat_l * (_SEQP // _CH)
        return pltpu.make_async_copy(
            table_sp.at[idx_v.at[pl.ds(bat_l * _PITCH + s * _CH, _CH)]],
            rows_v.at[b], gsems[b])

    def scatter_desc(c, b):
        return pltpu.make_async_copy(
            rows_v.at[b], rows_hbm.at[pl.ds(rb0 + c * _CH, _CH)], ssems[b])

    gather_desc(0, 0).start()
    gather_desc(1, 1).start()

    def outer(t, carry):
        for b in range(2):
            c = t * 2 + b
            gather_desc(c, b).wait()
            scatter_desc(c, b).start()

            @pl.when(c + 2 < ncht)
            def _():
                scatter_desc(c, b).wait()
                gather_desc(c + 2, b).start()
        return carry

    lax.fori_loop(0, ncht // 2, outer, None)

    for k in range(nk):
        pick_desc(k).wait()

    tail_m = lax.iota(jnp.int32, _L) < (_SEQ - 3 * _L)

    def accum(r, carry):
        for g in range(_PITCH // _L):
            off = r * _PITCH + g * _L
            valid = min(_L, _SEQ - g * _L)
            if valid <= 0:
                continue
            idxg = idx_v[pl.ds(off, _L)]
            picked = picks_v[pl.ds(off, _L)]
            if valid == _L:
                lsev = plsc.load_gather(lse_v, [idxg])
                acc_v[...] = acc_v[...] + (lsev - picked)
            else:
                lsev = plsc.load_gather(lse_v, [idxg], mask=tail_m)
                acc_v[...] = acc_v[...] + jnp.where(
                    tail_m, lsev - picked, jnp.zeros((_L,), jnp.float32))
        return carry

    lax.fori_loop(0, batw, accum, None)

    scatter_desc(ncht - 2, 0).wait()
    scatter_desc(ncht - 1, 1).wait()
    pltpu.sync_copy(acc_v, part_hbm.at[pl.ds(wid * _L, _L)])


def _sc_gather_loss(table, tflat, idx_p, tgt_p, lse_p, nbatch):
    batw = nbatch // _NW
    call = pl.kernel(
        _sc_body,
        out_type=[
            jax.ShapeDtypeStruct((nbatch * _SEQP, _VOCAB), jnp.float32),
            jax.ShapeDtypeStruct((_NW * _L,), jnp.float32),
        ],
        mesh=plsc.VectorSubcoreMesh(core_axis_name="c", subcore_axis_name="s"),
        compiler_params=pltpu.CompilerParams(
            use_tc_tiling_on_sc=False, needs_layout_passes=False),
        scratch_types=[
            pltpu.VMEM((2, _CH, _VOCAB), jnp.float32),
            pltpu.VMEM((batw * _PITCH,), jnp.int32),
            pltpu.VMEM((batw * _PITCH,), jnp.int32),
            pltpu.VMEM((batw * _PITCH,), jnp.int32),
            pltpu.VMEM((batw * _PITCH,), jnp.float32),
            pltpu.VMEM((_LSE_PAD,), jnp.float32),
            pltpu.VMEM((_L,), jnp.float32),
            pltpu.VMEM_SHARED((_VOCAB, _VOCAB), jnp.float32),
            pltpu.SemaphoreType.DMA,
            pltpu.SemaphoreType.DMA,
            pltpu.SemaphoreType.DMA,
            pltpu.SemaphoreType.DMA,
            pltpu.SemaphoreType.DMA,
        ],
    )
    return call(table, tflat, idx_p, tgt_p, lse_p)


def kernel(idx, targets, table):
    nbat, seq = idx.shape
    lse = _compute_lse(table)
    lse_p = jnp.pad(lse, (0, _LSE_PAD - _VOCAB))
    idx_p = jnp.pad(idx, ((0, 0), (0, _PITCH - _SEQ))).reshape(-1)
    tgt_p = jnp.pad(targets, ((0, 0), (0, _PITCH - _SEQ))).reshape(-1)
    tflat = jnp.pad(table, ((0, 0), (0, 1024 - _VOCAB))).reshape(-1)
    rows, partials = _sc_gather_loss(table, tflat, idx_p, tgt_p, lse_p, nbat)
    logits = rows.reshape(nbat, _SEQP, _VOCAB)[:, :_SEQ, :]
    loss = jnp.sum(partials) / jnp.float32(nbat * seq)
    return logits, loss


# speedup vs baseline: 2.0540x; 1.0646x over previous
import functools

import jax
import jax.numpy as jnp
from jax import lax
from jax.experimental import pallas as pl
from jax.experimental.pallas import tpu as pltpu
from jax.experimental.pallas import tpu_sc as plsc

_VOCAB = 1000
_LSE_PAD = 1008
_NC = 2
_NS = 16
_NW = _NC * _NS
_L = 16
_SEQ = 50
_SEQP = 56
_PITCH = 56
_CH = 16


def _lse_body(table_ref, out_ref):
    x = table_ref[...]
    m = jnp.max(x, axis=1)
    s = jnp.sum(jnp.exp(x - m[:, None]), axis=1)
    out_ref[...] = m + jnp.log(s)


def _compute_lse(table):
    return pl.pallas_call(
        _lse_body,
        out_shape=jax.ShapeDtypeStruct((table.shape[0],), jnp.float32),
    )(table)


def _sc_body(table_hbm, tflat_hbm, idx_hbm, tgt_hbm, lse_hbm,
             rows_hbm, part_hbm,
             rows_v, idx_v, tgt_v, fidx_v, picks_v, lse_v, acc_v, table_sp,
             g0, g1, s0, s1, psem):
    wid = lax.axis_index("s") * _NC + lax.axis_index("c")
    batw = (rows_hbm.shape[0] // _SEQP) // _NW
    npick = batw * _PITCH
    ib0 = wid * npick
    rb0 = wid * batw * _SEQP
    ncht = batw * _SEQP // _CH

    @pl.when(lax.axis_index("s") == 0)
    def _():
        pltpu.sync_copy(table_hbm, table_sp)

    pltpu.sync_copy(idx_hbm.at[pl.ds(ib0, npick)], idx_v.at[pl.ds(0, npick)])
    pltpu.sync_copy(tgt_hbm.at[pl.ds(ib0, npick)], tgt_v)
    pltpu.sync_copy(lse_hbm, lse_v)
    acc_v[...] = jnp.zeros((_L,), jnp.float32)
    plsc.subcore_barrier()

    def build(i, carry):
        off = i * _L
        fidx_v[pl.ds(off, _L)] = (
            idx_v[pl.ds(off, _L)] * 1024 + tgt_v[pl.ds(off, _L)])
        return carry

    lax.fori_loop(0, npick // _L, build, None)

    def pick_desc(k):
        return pltpu.make_async_copy(
            tflat_hbm.at[fidx_v.at[pl.ds(k * 128, 128)]],
            picks_v.at[pl.ds(k * 128, 128)], psem)

    nk = npick // 128
    for k in range(nk):
        pick_desc(k).start()

    gsems = (g0, g1)
    ssems = (s0, s1)

    def gather_desc(c, b):
        return pltpu.make_async_copy(
            table_sp.at[idx_v.at[pl.ds(c * _CH, _CH)]],
            rows_v.at[b], gsems[b])

    def scatter_desc(c, b):
        return pltpu.make_async_copy(
            rows_v.at[b], rows_hbm.at[pl.ds(rb0 + c * _CH, _CH)], ssems[b])

    gather_desc(0, 0).start()
    gather_desc(1, 1).start()

    def outer(t, carry):
        for b in range(2):
            c = t * 2 + b
            gather_desc(c, b).wait()
            scatter_desc(c, b).start()

            @pl.when(c + 2 < ncht)
            def _():
                scatter_desc(c, b).wait()
                gather_desc(c + 2, b).start()
        return carry

    lax.fori_loop(0, ncht // 2, outer, None)

    for k in range(nk):
        pick_desc(k).wait()

    tail_m = lax.iota(jnp.int32, _L) < (_SEQ - 3 * _L)

    def accum(r, carry):
        for g in range((_SEQ + _L - 1) // _L):
            off = r * _PITCH + g * _L
            valid = min(_L, _SEQ - g * _L)
            if valid <= 0:
                continue
            idxg = idx_v[pl.ds(off, _L)]
            picked = picks_v[pl.ds(off, _L)]
            if valid == _L:
                lsev = plsc.load_gather(lse_v, [idxg])
                acc_v[...] = acc_v[...] + (lsev - picked)
            else:
                lsev = plsc.load_gather(lse_v, [idxg], mask=tail_m)
                acc_v[...] = acc_v[...] + jnp.where(
                    tail_m, lsev - picked, jnp.zeros((_L,), jnp.float32))
        return carry

    lax.fori_loop(0, batw, accum, None)

    scatter_desc(ncht - 2, 0).wait()
    scatter_desc(ncht - 1, 1).wait()
    pltpu.sync_copy(acc_v, part_hbm.at[pl.ds(wid * _L, _L)])


def _sc_gather_loss(table, tflat, idx_p, tgt_p, lse_p, nbatch):
    batw = nbatch // _NW
    call = pl.kernel(
        _sc_body,
        out_type=[
            jax.ShapeDtypeStruct((nbatch * _SEQP, _VOCAB), jnp.float32),
            jax.ShapeDtypeStruct((_NW * _L,), jnp.float32),
        ],
        mesh=plsc.VectorSubcoreMesh(core_axis_name="c", subcore_axis_name="s"),
        compiler_params=pltpu.CompilerParams(
            use_tc_tiling_on_sc=False, needs_layout_passes=False),
        scratch_types=[
            pltpu.VMEM((2, _CH, _VOCAB), jnp.float32),
            pltpu.VMEM((batw * _PITCH + _L,), jnp.int32),
            pltpu.VMEM((batw * _PITCH,), jnp.int32),
            pltpu.VMEM((batw * _PITCH,), jnp.int32),
            pltpu.VMEM((batw * _PITCH + _L,), jnp.float32),
            pltpu.VMEM((_LSE_PAD,), jnp.float32),
            pltpu.VMEM((_L,), jnp.float32),
            pltpu.VMEM_SHARED((_VOCAB, _VOCAB), jnp.float32),
            pltpu.SemaphoreType.DMA,
            pltpu.SemaphoreType.DMA,
            pltpu.SemaphoreType.DMA,
            pltpu.SemaphoreType.DMA,
            pltpu.SemaphoreType.DMA,
        ],
    )
    return call(table, tflat, idx_p, tgt_p, lse_p)


def kernel(idx, targets, table):
    nbat, seq = idx.shape
    lse = _compute_lse(table)
    lse_p = jnp.pad(lse, (0, _LSE_PAD - _VOCAB))
    idx_p = jnp.pad(idx, ((0, 0), (0, _PITCH - _SEQ))).reshape(-1)
    tgt_p = jnp.pad(targets, ((0, 0), (0, _PITCH - _SEQ))).reshape(-1)
    tflat = jnp.pad(table, ((0, 0), (0, 1024 - _VOCAB))).reshape(-1)
    rows, partials = _sc_gather_loss(table, tflat, idx_p, tgt_p, lse_p, nbat)
    logits = rows.reshape(nbat, _SEQP, _VOCAB)[:, :_SEQ, :]
    loss = jnp.sum(partials) / jnp.float32(nbat * seq)
    return logits, loss
